# baseline (speedup 1.0000x reference)
import functools

import jax
import jax.numpy as jnp
from jax import lax
from jax.experimental import pallas as pl
from jax.experimental.pallas import tpu as pltpu
from jax.experimental.pallas import tpu_sc as plsc

N = 10000
E = 160000
D_IN = 128
D_OUT = 20
DP = 32
D_EDGE = 16
H = 64
NG = 64
N_ACT = 16
HID = 256
EPS = 1e-5

NC = 2
NS = 16
NW = NC * NS
CHUNK = 125
ROWS_W = E // NW // CHUNK
CHUNK_S = 40
ROWS_W_S = E // NW // CHUNK_S
N_PAD = 10240
N_TILE = N_PAD // NS

TE_STATS = 6400
TE_MSG = 1280
TN_FIN = 2000


def _stats_body(ea_ref, x_ref, root_ref, bias_ref, out_ref, xr_ref):
    i = pl.program_id(0)
    ea = ea_ref[...]
    aug = jnp.concatenate([ea, jnp.ones((ea.shape[0], 1), jnp.float32)], axis=1)
    part = lax.dot_general(aug, aug, (((0,), (0,)), ((), ())),
                           preferred_element_type=jnp.float32)

    @pl.when(i == 0)
    def _():
        out_ref[...] = part

    @pl.when(i > 0)
    def _():
        out_ref[...] = out_ref[...] + part

    xr_ref[...] = jnp.dot(x_ref[...], root_ref[...],
                          preferred_element_type=jnp.float32) + bias_ref[...]


def _edge_stats(edge_attr, x, root_p, bias_p):
    nsteps = E // TE_STATS
    return pl.pallas_call(
        _stats_body,
        grid=(nsteps,),
        in_specs=[
            pl.BlockSpec((TE_STATS, D_EDGE), lambda i: (i, 0)),
            pl.BlockSpec((N // nsteps, D_IN), lambda i: (i, 0)),
            pl.BlockSpec((D_IN, DP), lambda i: (0, 0)),
            pl.BlockSpec((1, DP), lambda i: (0, 0)),
        ],
        out_specs=(
            pl.BlockSpec((D_EDGE + 1, D_EDGE + 1), lambda i: (0, 0)),
            pl.BlockSpec((N // nsteps, DP), lambda i: (i, 0)),
        ),
        out_shape=(
            jax.ShapeDtypeStruct((D_EDGE + 1, D_EDGE + 1), jnp.float32),
            jax.ShapeDtypeStruct((N, DP), jnp.float32),
        ),
    )(edge_attr, x, root_p, bias_p)


def _gather_body(x_hbm, src_hbm, out_hbm, idx_v, rows_v, sem):
    c = lax.axis_index("c")
    s = lax.axis_index("s")
    wid = s * NC + c
    pltpu.sync_copy(src_hbm.at[pl.ds(wid * ROWS_W, ROWS_W)], idx_v)

    def body(j, carry):
        pltpu.async_copy(x_hbm.at[idx_v.at[j]], rows_v, sem).wait()
        pltpu.sync_copy(
            rows_v, out_hbm.at[pl.ds(wid * ROWS_W * CHUNK + j * CHUNK, CHUNK)])
        return carry

    lax.fori_loop(0, ROWS_W, body, 0)


def _gather_rows(x, src2d):
    kfn = pl.kernel(
        _gather_body,
        out_type=jax.ShapeDtypeStruct((E, D_IN), jnp.float32),
        mesh=plsc.VectorSubcoreMesh(core_axis_name="c", subcore_axis_name="s"),
        compiler_params=pltpu.CompilerParams(use_tc_tiling_on_sc=False),
        scratch_types=[
            pltpu.VMEM((ROWS_W, CHUNK), jnp.int32),
            pltpu.VMEM((CHUNK, D_IN), jnp.float32),
            pltpu.SemaphoreType.DMA,
        ],
    )
    return kfn(x, src2d)


def _msg_body(ea_ref, xs_ref, W1_ref, a_ref, c_ref, W2f_ref, S_ref, b2r_ref,
              m_ref):
    ea = ea_ref[...]
    xs = xs_ref[...]
    xsb = xs.astype(jnp.bfloat16)
    h = jnp.dot(ea, W1_ref[...], preferred_element_type=jnp.float32)
    h = jnp.maximum(h * a_ref[...] + c_ref[...], 0.0)
    h2 = jnp.concatenate([h, h], axis=1)
    m = jnp.dot(xsb, b2r_ref[...], preferred_element_type=jnp.float32)
    for j in range(D_OUT // 2):
        Gj = jnp.dot(xsb, W2f_ref[:, j * D_IN:(j + 1) * D_IN],
                     preferred_element_type=jnp.float32)
        prodj = (Gj * h2).astype(jnp.bfloat16)
        m = m + jnp.dot(prodj, S_ref[j * D_IN:(j + 1) * D_IN, :],
                        preferred_element_type=jnp.float32)
    m_ref[...] = m


def _messages(edge_attr, x_src, W1, a_vec, c_vec, W2flat, S, b2r):
    full = lambda r, c: pl.BlockSpec((r, c), lambda i: (0, 0))
    return pl.pallas_call(
        _msg_body,
        grid=(E // TE_MSG,),
        in_specs=[
            pl.BlockSpec((TE_MSG, D_EDGE), lambda i: (i, 0)),
            pl.BlockSpec((TE_MSG, D_IN), lambda i: (i, 0)),
            full(D_EDGE, H),
            full(1, H),
            full(1, H),
            full(D_IN, D_OUT * H),
            full(D_OUT * H, DP),
            full(D_IN, DP),
        ],
        out_specs=pl.BlockSpec((TE_MSG, DP), lambda i: (i, 0)),
        out_shape=jax.ShapeDtypeStruct((E, DP), jnp.float32),
    )(edge_attr, x_src, W1, a_vec, c_vec, W2flat, S, b2r)


def _scatter_body(m_hbm, dst_hbm, zeros_hbm, out_hbm, idx_v, mbuf, aggr_sh,
                  sem):
    c = lax.axis_index("c")
    s = lax.axis_index("s")
    wid = s * NC + c
    pltpu.sync_copy(zeros_hbm.at[pl.ds(s * N_TILE, N_TILE)],
                    aggr_sh.at[pl.ds(s * N_TILE, N_TILE)])
    pltpu.sync_copy(dst_hbm.at[pl.ds(wid * ROWS_W, ROWS_W)], idx_v)
    plsc.subcore_barrier()

    def body(j, carry):
        pltpu.sync_copy(
            m_hbm.at[pl.ds(wid * ROWS_W * CHUNK + j * CHUNK, CHUNK)], mbuf)
        pltpu.sync_copy(mbuf, aggr_sh.at[idx_v.at[j]], add=True)
        return carry

    lax.fori_loop(0, ROWS_W, body, 0)
    plsc.subcore_barrier()
    pltpu.sync_copy(aggr_sh.at[pl.ds(s * N_TILE, N_TILE)],
                    out_hbm.at[c, pl.ds(s * N_TILE, N_TILE)])


def _scatter_add(m, dst2d, zeros):
    kfn = pl.kernel(
        _scatter_body,
        out_type=jax.ShapeDtypeStruct((NC, N_PAD, DP), jnp.float32),
        mesh=plsc.VectorSubcoreMesh(core_axis_name="c", subcore_axis_name="s"),
        compiler_params=pltpu.CompilerParams(use_tc_tiling_on_sc=False),
        scratch_types=[
            pltpu.VMEM((ROWS_W, CHUNK), jnp.int32),
            pltpu.VMEM((CHUNK, DP), jnp.float32),
            pltpu.VMEM_SHARED((N_PAD, DP), jnp.float32),
            pltpu.SemaphoreType.DMA,
        ],
    )
    return kfn(m, dst2d, zeros)


def _final_body(p0_ref, p1_ref, xr_ref, b_ref, A1_ref, bA1_ref, A2_ref,
                bA2_ref, out_ref, acc_ref):
    i = pl.program_id(0)
    out32 = p0_ref[...] + p1_ref[...] + xr_ref[...]
    lanes = lax.broadcasted_iota(jnp.int32, (TN_FIN, DP), 1)
    out_aug = out32 + (lanes == D_OUT).astype(jnp.float32)
    gids = lax.broadcasted_iota(jnp.int32, (TN_FIN, NG), 1)
    onehot = (b_ref[...] == gids).astype(jnp.float32)
    part = lax.dot_general(onehot, out_aug, (((0,), (0,)), ((), ())),
                           preferred_element_type=jnp.float32)

    @pl.when(i == 0)
    def _():
        acc_ref[...] = part

    @pl.when(i > 0)
    def _():
        acc_ref[...] = acc_ref[...] + part

    @pl.when(i == (N // TN_FIN) - 1)
    def _():
        P = acc_ref[...]
        cnt = P[:, D_OUT:D_OUT + 1]
        pooled = P / jnp.maximum(cnt, 1.0)
        z = jnp.maximum(
            jnp.dot(pooled, A1_ref[...], preferred_element_type=jnp.float32)
            + bA1_ref[...], 0.0)
        out_ref[...] = jnp.dot(
            z, A2_ref[...], preferred_element_type=jnp.float32) + bA2_ref[...]


def _finalize(p0, p1, xroot, batch2d, A1p, bA1, A2, bA2):
    full = lambda r, c: pl.BlockSpec((r, c), lambda i: (0, 0))
    return pl.pallas_call(
        _final_body,
        grid=(N // TN_FIN,),
        in_specs=[
            pl.BlockSpec((TN_FIN, DP), lambda i: (i, 0)),
            pl.BlockSpec((TN_FIN, DP), lambda i: (i, 0)),
            pl.BlockSpec((TN_FIN, DP), lambda i: (i, 0)),
            pl.BlockSpec((TN_FIN, 1), lambda i: (i, 0)),
            full(DP, HID),
            full(1, HID),
            full(HID, N_ACT),
            full(1, N_ACT),
        ],
        out_specs=pl.BlockSpec((NG, N_ACT), lambda i: (0, 0)),
        out_shape=jax.ShapeDtypeStruct((NG, N_ACT), jnp.float32),
        scratch_shapes=[pltpu.VMEM((NG, DP), jnp.float32)],
    )(p0, p1, xroot, batch2d, A1p, bA1, A2, bA2)


def kernel(x, edge_index, edge_attr, batch, W1, b1, gamma, beta, W2, b2, root,
           bias, A1, bA1, A2, bA2):
    f32 = jnp.float32
    src2d = edge_index[0].reshape(NW * ROWS_W, CHUNK)
    dst2d = edge_index[1].reshape(NW * ROWS_W, CHUNK)
    root_p = jnp.pad(root, ((0, 0), (0, DP - D_OUT)))
    bias_p = jnp.pad(bias, (0, DP - D_OUT)).reshape(1, DP)

    C_aug, xroot = _edge_stats(edge_attr, x, root_p, bias_p)
    s_vec = C_aug[D_EDGE, :D_EDGE]
    Cm = C_aug[:D_EDGE, :D_EDGE]
    mu = (s_vec / E) @ W1 + b1
    Eh2 = (jnp.einsum("ij,ik,kj->j", W1, Cm, W1)
           + 2.0 * b1 * (s_vec @ W1)) / E + b1 * b1
    var = Eh2 - mu * mu
    inv = gamma * lax.rsqrt(var + EPS)
    a_vec = inv.reshape(1, H)
    c_vec = ((b1 - mu) * inv + beta).reshape(1, H)

    W2flat = W2.T.reshape(D_IN, D_OUT * H).astype(jnp.bfloat16)
    col = jnp.arange(D_OUT * H, dtype=jnp.int32)[:, None]
    S = (col // H == jnp.arange(DP, dtype=jnp.int32)[None, :]).astype(
        jnp.bfloat16)
    b2r = jnp.pad(b2.reshape(D_IN, D_OUT),
                  ((0, 0), (0, DP - D_OUT))).astype(jnp.bfloat16)
    A1p = jnp.pad(A1, ((0, DP - D_OUT), (0, 0)))

    x_src = _gather_rows(x, src2d)

    m = _messages(edge_attr, x_src, W1, a_vec, c_vec, W2flat, S, b2r)

    partials = _scatter_add(m, dst2d, jnp.zeros((N_PAD, DP), f32))

    return _finalize(partials[0, :N], partials[1, :N], xroot,
                     batch.reshape(N, 1), A1p, bA1.reshape(1, HID), A2,
                     bA2.reshape(1, N_ACT))


# speedup vs baseline: 1.0580x; 1.0580x over previous
import functools

import jax
import jax.numpy as jnp
from jax import lax
from jax.experimental import pallas as pl
from jax.experimental.pallas import tpu as pltpu
from jax.experimental.pallas import tpu_sc as plsc

N = 10000
E = 160000
D_IN = 128
D_OUT = 20
DP = 32
D_EDGE = 16
H = 64
NG = 64
N_ACT = 16
HID = 256
EPS = 1e-5

NC = 2
NS = 16
NW = NC * NS
CHUNK = 125
ROWS_W = E // NW // CHUNK
CHUNK_S = 40
ROWS_W_S = E // NW // CHUNK_S
N_PAD = 10240
N_TILE = N_PAD // NS

TE_STATS = 6400
TE_MSG = 1280
TN_FIN = 2000


def _stats_body(eat_ref, x_ref, root_ref, bias_ref, out_ref, xr_ref):
    i = pl.program_id(0)
    eat = eat_ref[...]
    aug = jnp.concatenate(
        [eat, jnp.ones((1, eat.shape[1]), jnp.float32)], axis=0)
    part = lax.dot_general(aug, aug, (((1,), (1,)), ((), ())),
                           preferred_element_type=jnp.float32)

    @pl.when(i == 0)
    def _():
        out_ref[...] = part

    @pl.when(i > 0)
    def _():
        out_ref[...] = out_ref[...] + part

    xr_ref[...] = jnp.dot(x_ref[...], root_ref[...],
                          preferred_element_type=jnp.float32) + bias_ref[...]


def _edge_stats(eat, x, root_p, bias_p):
    nsteps = E // TE_STATS
    return pl.pallas_call(
        _stats_body,
        grid=(nsteps,),
        in_specs=[
            pl.BlockSpec((D_EDGE, TE_STATS), lambda i: (0, i)),
            pl.BlockSpec((N // nsteps, D_IN), lambda i: (i, 0)),
            pl.BlockSpec((D_IN, DP), lambda i: (0, 0)),
            pl.BlockSpec((1, DP), lambda i: (0, 0)),
        ],
        out_specs=(
            pl.BlockSpec((D_EDGE + 1, D_EDGE + 1), lambda i: (0, 0)),
            pl.BlockSpec((N // nsteps, DP), lambda i: (i, 0)),
        ),
        out_shape=(
            jax.ShapeDtypeStruct((D_EDGE + 1, D_EDGE + 1), jnp.float32),
            jax.ShapeDtypeStruct((N, DP), jnp.float32),
        ),
    )(eat, x, root_p, bias_p)


def _gather_body(x_hbm, src_hbm, out_hbm, idx_v, rows_v, sem):
    c = lax.axis_index("c")
    s = lax.axis_index("s")
    wid = s * NC + c
    pltpu.sync_copy(src_hbm.at[pl.ds(wid * ROWS_W, ROWS_W)], idx_v)

    def body(j, carry):
        pltpu.async_copy(x_hbm.at[idx_v.at[j]], rows_v, sem).wait()
        pltpu.sync_copy(
            rows_v, out_hbm.at[pl.ds(wid * ROWS_W * CHUNK + j * CHUNK, CHUNK)])
        return carry

    lax.fori_loop(0, ROWS_W, body, 0)


def _gather_rows(x, src2d):
    kfn = pl.kernel(
        _gather_body,
        out_type=jax.ShapeDtypeStruct((E, D_IN), jnp.float32),
        mesh=plsc.VectorSubcoreMesh(core_axis_name="c", subcore_axis_name="s"),
        compiler_params=pltpu.CompilerParams(use_tc_tiling_on_sc=False),
        scratch_types=[
            pltpu.VMEM((ROWS_W, CHUNK), jnp.int32),
            pltpu.VMEM((CHUNK, D_IN), jnp.float32),
            pltpu.SemaphoreType.DMA,
        ],
    )
    return kfn(x, src2d)


def _msg_body(eat_ref, xs_ref, W1_ref, a_ref, c_ref, W2f_ref, S_ref, b2r_ref,
              m_ref):
    eat = eat_ref[...]
    xs = xs_ref[...]
    xsb = xs.astype(jnp.bfloat16)
    h = lax.dot_general(eat, W1_ref[...], (((0,), (0,)), ((), ())),
                        preferred_element_type=jnp.float32)
    h = jnp.maximum(h * a_ref[...] + c_ref[...], 0.0)
    h2 = jnp.concatenate([h, h], axis=1)
    m = jnp.dot(xsb, b2r_ref[...], preferred_element_type=jnp.float32)
    for j in range(D_OUT // 2):
        Gj = jnp.dot(xsb, W2f_ref[:, j * D_IN:(j + 1) * D_IN],
                     preferred_element_type=jnp.float32)
        prodj = (Gj * h2).astype(jnp.bfloat16)
        m = m + jnp.dot(prodj, S_ref[j * D_IN:(j + 1) * D_IN, :],
                        preferred_element_type=jnp.float32)
    m_ref[...] = m


def _messages(eat, x_src, W1, a_vec, c_vec, W2flat, S, b2r):
    full = lambda r, c: pl.BlockSpec((r, c), lambda i: (0, 0))
    return pl.pallas_call(
        _msg_body,
        grid=(E // TE_MSG,),
        in_specs=[
            pl.BlockSpec((D_EDGE, TE_MSG), lambda i: (0, i)),
            pl.BlockSpec((TE_MSG, D_IN), lambda i: (i, 0)),
            full(D_EDGE, H),
            full(1, H),
            full(1, H),
            full(D_IN, D_OUT * H),
            full(D_OUT * H, DP),
            full(D_IN, DP),
        ],
        out_specs=pl.BlockSpec((TE_MSG, DP), lambda i: (i, 0)),
        out_shape=jax.ShapeDtypeStruct((E, DP), jnp.float32),
    )(eat, x_src, W1, a_vec, c_vec, W2flat, S, b2r)


def _scatter_body(m_hbm, dst_hbm, zeros_hbm, out_hbm, idx_v, mbuf, aggr_sh,
                  sem):
    c = lax.axis_index("c")
    s = lax.axis_index("s")
    wid = s * NC + c
    pltpu.sync_copy(zeros_hbm.at[pl.ds(s * N_TILE, N_TILE)],
                    aggr_sh.at[pl.ds(s * N_TILE, N_TILE)])
    pltpu.sync_copy(dst_hbm.at[pl.ds(wid * ROWS_W, ROWS_W)], idx_v)
    plsc.subcore_barrier()

    def body(j, carry):
        pltpu.sync_copy(
            m_hbm.at[pl.ds(wid * ROWS_W * CHUNK + j * CHUNK, CHUNK)], mbuf)
        pltpu.sync_copy(mbuf, aggr_sh.at[idx_v.at[j]], add=True)
        return carry

    lax.fori_loop(0, ROWS_W, body, 0)
    plsc.subcore_barrier()
    pltpu.sync_copy(aggr_sh.at[pl.ds(s * N_TILE, N_TILE)],
                    out_hbm.at[c, pl.ds(s * N_TILE, N_TILE)])


def _scatter_add(m, dst2d, zeros):
    kfn = pl.kernel(
        _scatter_body,
        out_type=jax.ShapeDtypeStruct((NC, N_PAD, DP), jnp.float32),
        mesh=plsc.VectorSubcoreMesh(core_axis_name="c", subcore_axis_name="s"),
        compiler_params=pltpu.CompilerParams(use_tc_tiling_on_sc=False),
        scratch_types=[
            pltpu.VMEM((ROWS_W, CHUNK), jnp.int32),
            pltpu.VMEM((CHUNK, DP), jnp.float32),
            pltpu.VMEM_SHARED((N_PAD, DP), jnp.float32),
            pltpu.SemaphoreType.DMA,
        ],
    )
    return kfn(m, dst2d, zeros)


def _final_body(p0_ref, p1_ref, xr_ref, b_ref, A1_ref, bA1_ref, A2_ref,
                bA2_ref, out_ref, acc_ref):
    i = pl.program_id(0)
    out32 = p0_ref[...] + p1_ref[...] + xr_ref[...]
    lanes = lax.broadcasted_iota(jnp.int32, (TN_FIN, DP), 1)
    out_aug = out32 + (lanes == D_OUT).astype(jnp.float32)
    gids = lax.broadcasted_iota(jnp.int32, (TN_FIN, NG), 1)
    onehot = (b_ref[...] == gids).astype(jnp.float32)
    part = lax.dot_general(onehot, out_aug, (((0,), (0,)), ((), ())),
                           preferred_element_type=jnp.float32)

    @pl.when(i == 0)
    def _():
        acc_ref[...] = part

    @pl.when(i > 0)
    def _():
        acc_ref[...] = acc_ref[...] + part

    @pl.when(i == (N // TN_FIN) - 1)
    def _():
        P = acc_ref[...]
        cnt = P[:, D_OUT:D_OUT + 1]
        pooled = P / jnp.maximum(cnt, 1.0)
        z = jnp.maximum(
            jnp.dot(pooled, A1_ref[...], preferred_element_type=jnp.float32)
            + bA1_ref[...], 0.0)
        out_ref[...] = jnp.dot(
            z, A2_ref[...], preferred_element_type=jnp.float32) + bA2_ref[...]


def _finalize(p0, p1, xroot, batch2d, A1p, bA1, A2, bA2):
    full = lambda r, c: pl.BlockSpec((r, c), lambda i: (0, 0))
    return pl.pallas_call(
        _final_body,
        grid=(N // TN_FIN,),
        in_specs=[
            pl.BlockSpec((TN_FIN, DP), lambda i: (i, 0)),
            pl.BlockSpec((TN_FIN, DP), lambda i: (i, 0)),
            pl.BlockSpec((TN_FIN, DP), lambda i: (i, 0)),
            pl.BlockSpec((TN_FIN, 1), lambda i: (i, 0)),
            full(DP, HID),
            full(1, HID),
            full(HID, N_ACT),
            full(1, N_ACT),
        ],
        out_specs=pl.BlockSpec((NG, N_ACT), lambda i: (0, 0)),
        out_shape=jax.ShapeDtypeStruct((NG, N_ACT), jnp.float32),
        scratch_shapes=[pltpu.VMEM((NG, DP), jnp.float32)],
    )(p0, p1, xroot, batch2d, A1p, bA1, A2, bA2)


def kernel(x, edge_index, edge_attr, batch, W1, b1, gamma, beta, W2, b2, root,
           bias, A1, bA1, A2, bA2):
    f32 = jnp.float32
    src2d = edge_index[0].reshape(NW * ROWS_W, CHUNK)
    dst2d = edge_index[1].reshape(NW * ROWS_W, CHUNK)
    root_p = jnp.pad(root, ((0, 0), (0, DP - D_OUT)))
    bias_p = jnp.pad(bias, (0, DP - D_OUT)).reshape(1, DP)

    eat = edge_attr.T
    C_aug, xroot = _edge_stats(eat, x, root_p, bias_p)
    s_vec = C_aug[D_EDGE, :D_EDGE]
    Cm = C_aug[:D_EDGE, :D_EDGE]
    mu = (s_vec / E) @ W1 + b1
    Eh2 = (jnp.einsum("ij,ik,kj->j", W1, Cm, W1)
           + 2.0 * b1 * (s_vec @ W1)) / E + b1 * b1
    var = Eh2 - mu * mu
    inv = gamma * lax.rsqrt(var + EPS)
    a_vec = inv.reshape(1, H)
    c_vec = ((b1 - mu) * inv + beta).reshape(1, H)

    W2flat = W2.T.reshape(D_IN, D_OUT * H).astype(jnp.bfloat16)
    col = jnp.arange(D_OUT * H, dtype=jnp.int32)[:, None]
    S = (col // H == jnp.arange(DP, dtype=jnp.int32)[None, :]).astype(
        jnp.bfloat16)
    b2r = jnp.pad(b2.reshape(D_IN, D_OUT),
                  ((0, 0), (0, DP - D_OUT))).astype(jnp.bfloat16)
    A1p = jnp.pad(A1, ((0, DP - D_OUT), (0, 0)))

    x_src = _gather_rows(x, src2d)

    m4 = _messages(eat, x_src, W1, a_vec, c_vec, W2flat, S, b2r)

    partials = _scatter_add(m4, dst2d, jnp.zeros((N_PAD, DP), f32))

    return _finalize(partials[0, :N], partials[1, :N], xroot,
                     batch.reshape(N, 1), A1p, bA1.reshape(1, HID), A2,
                     bA2.reshape(1, N_ACT))


# speedup vs baseline: 1.1078x; 1.0471x over previous
import functools

import jax
import jax.numpy as jnp
from jax import lax
from jax.experimental import pallas as pl
from jax.experimental.pallas import tpu as pltpu
from jax.experimental.pallas import tpu_sc as plsc

N = 10000
E = 160000
D_IN = 128
D_OUT = 20
DP = 32
D_EDGE = 16
H = 64
NG = 64
N_ACT = 16
HID = 256
EPS = 1e-5

NC = 2
NS = 16
NW = NC * NS
CHUNK = 125
ROWS_W = E // NW // CHUNK
CHUNK_S = 40
ROWS_W_S = E // NW // CHUNK_S
N_PAD = 10240
N_TILE = N_PAD // NS

TE_STATS = 6400
TE_MSG = 3200
TN_FIN = 2000


def _stats_body(eat_ref, x_ref, root_ref, bias_ref, out_ref, xr_ref):
    i = pl.program_id(0)
    eat = eat_ref[...]
    aug = jnp.concatenate(
        [eat, jnp.ones((1, eat.shape[1]), jnp.float32)], axis=0)
    part = lax.dot_general(aug, aug, (((1,), (1,)), ((), ())),
                           preferred_element_type=jnp.float32)

    @pl.when(i == 0)
    def _():
        out_ref[...] = part

    @pl.when(i > 0)
    def _():
        out_ref[...] = out_ref[...] + part

    xr_ref[...] = jnp.dot(x_ref[...], root_ref[...],
                          preferred_element_type=jnp.float32) + bias_ref[...]


def _edge_stats(eat, x, root_p, bias_p):
    nsteps = E // TE_STATS
    return pl.pallas_call(
        _stats_body,
        grid=(nsteps,),
        in_specs=[
            pl.BlockSpec((D_EDGE, TE_STATS), lambda i: (0, i)),
            pl.BlockSpec((N // nsteps, D_IN), lambda i: (i, 0)),
            pl.BlockSpec((D_IN, DP), lambda i: (0, 0)),
            pl.BlockSpec((1, DP), lambda i: (0, 0)),
        ],
        out_specs=(
            pl.BlockSpec((D_EDGE + 1, D_EDGE + 1), lambda i: (0, 0)),
            pl.BlockSpec((N // nsteps, DP), lambda i: (i, 0)),
        ),
        out_shape=(
            jax.ShapeDtypeStruct((D_EDGE + 1, D_EDGE + 1), jnp.float32),
            jax.ShapeDtypeStruct((N, DP), jnp.float32),
        ),
    )(eat, x, root_p, bias_p)


def _gather_body(x_hbm, src_hbm, out_hbm, idx_v, rows_v, sem):
    c = lax.axis_index("c")
    s = lax.axis_index("s")
    wid = s * NC + c
    pltpu.sync_copy(src_hbm.at[pl.ds(wid * ROWS_W, ROWS_W)], idx_v)

    def body(j, carry):
        pltpu.async_copy(x_hbm.at[idx_v.at[j]], rows_v, sem).wait()
        pltpu.sync_copy(
            rows_v, out_hbm.at[pl.ds(wid * ROWS_W * CHUNK + j * CHUNK, CHUNK)])
        return carry

    lax.fori_loop(0, ROWS_W, body, 0)


def _gather_rows(x, src2d):
    kfn = pl.kernel(
        _gather_body,
        out_type=jax.ShapeDtypeStruct((E, D_IN), jnp.float32),
        mesh=plsc.VectorSubcoreMesh(core_axis_name="c", subcore_axis_name="s"),
        compiler_params=pltpu.CompilerParams(use_tc_tiling_on_sc=False),
        scratch_types=[
            pltpu.VMEM((ROWS_W, CHUNK), jnp.int32),
            pltpu.VMEM((CHUNK, D_IN), jnp.float32),
            pltpu.SemaphoreType.DMA,
        ],
    )
    return kfn(x, src2d)


def _msg_body(eat_ref, xs_ref, W1_ref, a_ref, c_ref, W2f_ref, S_ref, b2r_ref,
              m_ref):
    eat = eat_ref[...]
    xs = xs_ref[...]
    xsb = xs.astype(jnp.bfloat16)
    h = lax.dot_general(eat, W1_ref[...], (((0,), (0,)), ((), ())),
                        preferred_element_type=jnp.float32)
    h = jnp.maximum(h * a_ref[...] + c_ref[...], 0.0)
    h2 = jnp.concatenate([h, h], axis=1)
    m = jnp.dot(xsb, b2r_ref[...], preferred_element_type=jnp.float32)
    for j in range(D_OUT // 2):
        Gj = jnp.dot(xsb, W2f_ref[:, j * D_IN:(j + 1) * D_IN],
                     preferred_element_type=jnp.float32)
        prodj = (Gj * h2).astype(jnp.bfloat16)
        m = m + jnp.dot(prodj, S_ref[j * D_IN:(j + 1) * D_IN, :],
                        preferred_element_type=jnp.float32)
    m_ref[...] = m


def _messages(eat, x_src, W1, a_vec, c_vec, W2flat, S, b2r):
    full = lambda r, c: pl.BlockSpec((r, c), lambda i: (0, 0))
    return pl.pallas_call(
        _msg_body,
        grid=(E // TE_MSG,),
        in_specs=[
            pl.BlockSpec((D_EDGE, TE_MSG), lambda i: (0, i)),
            pl.BlockSpec((TE_MSG, D_IN), lambda i: (i, 0)),
            full(D_EDGE, H),
            full(1, H),
            full(1, H),
            full(D_IN, D_OUT * H),
            full(D_OUT * H, DP),
            full(D_IN, DP),
        ],
        out_specs=pl.BlockSpec((TE_MSG, DP), lambda i: (i, 0)),
        out_shape=jax.ShapeDtypeStruct((E, DP), jnp.float32),
    )(eat, x_src, W1, a_vec, c_vec, W2flat, S, b2r)


def _scatter_body(m_hbm, dst_hbm, zeros_hbm, out_hbm, idx_v, mbuf, aggr_sh,
                  sem):
    c = lax.axis_index("c")
    s = lax.axis_index("s")
    wid = s * NC + c
    pltpu.sync_copy(zeros_hbm.at[pl.ds(s * N_TILE, N_TILE)],
                    aggr_sh.at[pl.ds(s * N_TILE, N_TILE)])
    pltpu.sync_copy(dst_hbm.at[pl.ds(wid * ROWS_W, ROWS_W)], idx_v)
    plsc.subcore_barrier()

    def body(j, carry):
        pltpu.sync_copy(
            m_hbm.at[pl.ds(wid * ROWS_W * CHUNK + j * CHUNK, CHUNK)], mbuf)
        pltpu.sync_copy(mbuf, aggr_sh.at[idx_v.at[j]], add=True)
        return carry

    lax.fori_loop(0, ROWS_W, body, 0)
    plsc.subcore_barrier()
    pltpu.sync_copy(aggr_sh.at[pl.ds(s * N_TILE, N_TILE)],
                    out_hbm.at[c, pl.ds(s * N_TILE, N_TILE)])


def _scatter_add(m, dst2d, zeros):
    kfn = pl.kernel(
        _scatter_body,
        out_type=jax.ShapeDtypeStruct((NC, N_PAD, DP), jnp.float32),
        mesh=plsc.VectorSubcoreMesh(core_axis_name="c", subcore_axis_name="s"),
        compiler_params=pltpu.CompilerParams(use_tc_tiling_on_sc=False),
        scratch_types=[
            pltpu.VMEM((ROWS_W, CHUNK), jnp.int32),
            pltpu.VMEM((CHUNK, DP), jnp.float32),
            pltpu.VMEM_SHARED((N_PAD, DP), jnp.float32),
            pltpu.SemaphoreType.DMA,
        ],
    )
    return kfn(m, dst2d, zeros)


def _final_body(p0_ref, p1_ref, xr_ref, b_ref, A1_ref, bA1_ref, A2_ref,
                bA2_ref, out_ref, acc_ref):
    i = pl.program_id(0)
    out32 = p0_ref[...] + p1_ref[...] + xr_ref[...]
    lanes = lax.broadcasted_iota(jnp.int32, (TN_FIN, DP), 1)
    out_aug = out32 + (lanes == D_OUT).astype(jnp.float32)
    gids = lax.broadcasted_iota(jnp.int32, (TN_FIN, NG), 1)
    onehot = (b_ref[...] == gids).astype(jnp.float32)
    part = lax.dot_general(onehot, out_aug, (((0,), (0,)), ((), ())),
                           preferred_element_type=jnp.float32)

    @pl.when(i == 0)
    def _():
        acc_ref[...] = part

    @pl.when(i > 0)
    def _():
        acc_ref[...] = acc_ref[...] + part

    @pl.when(i == (N // TN_FIN) - 1)
    def _():
        P = acc_ref[...]
        cnt = P[:, D_OUT:D_OUT + 1]
        pooled = P / jnp.maximum(cnt, 1.0)
        z = jnp.maximum(
            jnp.dot(pooled, A1_ref[...], preferred_element_type=jnp.float32)
            + bA1_ref[...], 0.0)
        out_ref[...] = jnp.dot(
            z, A2_ref[...], preferred_element_type=jnp.float32) + bA2_ref[...]


def _finalize(p0, p1, xroot, batch2d, A1p, bA1, A2, bA2):
    full = lambda r, c: pl.BlockSpec((r, c), lambda i: (0, 0))
    return pl.pallas_call(
        _final_body,
        grid=(N // TN_FIN,),
        in_specs=[
            pl.BlockSpec((TN_FIN, DP), lambda i: (i, 0)),
            pl.BlockSpec((TN_FIN, DP), lambda i: (i, 0)),
            pl.BlockSpec((TN_FIN, DP), lambda i: (i, 0)),
            pl.BlockSpec((TN_FIN, 1), lambda i: (i, 0)),
            full(DP, HID),
            full(1, HID),
            full(HID, N_ACT),
            full(1, N_ACT),
        ],
        out_specs=pl.BlockSpec((NG, N_ACT), lambda i: (0, 0)),
        out_shape=jax.ShapeDtypeStruct((NG, N_ACT), jnp.float32),
        scratch_shapes=[pltpu.VMEM((NG, DP), jnp.float32)],
    )(p0, p1, xroot, batch2d, A1p, bA1, A2, bA2)


def kernel(x, edge_index, edge_attr, batch, W1, b1, gamma, beta, W2, b2, root,
           bias, A1, bA1, A2, bA2):
    f32 = jnp.float32
    src2d = edge_index[0].reshape(NW * ROWS_W, CHUNK)
    dst2d = edge_index[1].reshape(NW * ROWS_W, CHUNK)
    root_p = jnp.pad(root, ((0, 0), (0, DP - D_OUT)))
    bias_p = jnp.pad(bias, (0, DP - D_OUT)).reshape(1, DP)

    eat = edge_attr.T
    C_aug, xroot = _edge_stats(eat, x, root_p, bias_p)
    s_vec = C_aug[D_EDGE, :D_EDGE]
    Cm = C_aug[:D_EDGE, :D_EDGE]
    mu = (s_vec / E) @ W1 + b1
    Eh2 = (jnp.einsum("ij,ik,kj->j", W1, Cm, W1)
           + 2.0 * b1 * (s_vec @ W1)) / E + b1 * b1
    var = Eh2 - mu * mu
    inv = gamma * lax.rsqrt(var + EPS)
    a_vec = inv.reshape(1, H)
    c_vec = ((b1 - mu) * inv + beta).reshape(1, H)

    W2flat = W2.T.reshape(D_IN, D_OUT * H).astype(jnp.bfloat16)
    col = jnp.arange(D_OUT * H, dtype=jnp.int32)[:, None]
    S = (col // H == jnp.arange(DP, dtype=jnp.int32)[None, :]).astype(
        jnp.bfloat16)
    b2r = jnp.pad(b2.reshape(D_IN, D_OUT),
                  ((0, 0), (0, DP - D_OUT))).astype(jnp.bfloat16)
    A1p = jnp.pad(A1, ((0, DP - D_OUT), (0, 0)))

    x_src = _gather_rows(x, src2d)

    m4 = _messages(eat, x_src, W1, a_vec, c_vec, W2flat, S, b2r)

    partials = _scatter_add(m4, dst2d, jnp.zeros((N_PAD, DP), f32))

    return _finalize(partials[0, :N], partials[1, :N], xroot,
                     batch.reshape(N, 1), A1p, bA1.reshape(1, HID), A2,
                     bA2.reshape(1, N_ACT))


# speedup vs baseline: 1.1233x; 1.0139x over previous
import functools

import jax
import jax.numpy as jnp
from jax import lax
from jax.experimental import pallas as pl
from jax.experimental.pallas import tpu as pltpu
from jax.experimental.pallas import tpu_sc as plsc

N = 10000
E = 160000
D_IN = 128
D_OUT = 20
DP = 32
D_EDGE = 16
H = 64
NG = 64
N_ACT = 16
HID = 256
EPS = 1e-5

NC = 2
NS = 16
NW = NC * NS
CHUNK = 125
ROWS_W = E // NW // CHUNK
CHUNK_S = 40
ROWS_W_S = E // NW // CHUNK_S
N_PAD = 10240
N_TILE = N_PAD // NS

TE_STATS = 6400
TE_MSG = 6400
TN_FIN = 2000


def _stats_body(eat_ref, x_ref, root_ref, bias_ref, out_ref, xr_ref):
    i = pl.program_id(0)
    eat = eat_ref[...]
    aug = jnp.concatenate(
        [eat, jnp.ones((1, eat.shape[1]), jnp.float32)], axis=0)
    part = lax.dot_general(aug, aug, (((1,), (1,)), ((), ())),
                           preferred_element_type=jnp.float32)

    @pl.when(i == 0)
    def _():
        out_ref[...] = part

    @pl.when(i > 0)
    def _():
        out_ref[...] = out_ref[...] + part

    xr_ref[...] = jnp.dot(x_ref[...], root_ref[...],
                          preferred_element_type=jnp.float32) + bias_ref[...]


def _edge_stats(eat, x, root_p, bias_p):
    nsteps = E // TE_STATS
    return pl.pallas_call(
        _stats_body,
        grid=(nsteps,),
        in_specs=[
            pl.BlockSpec((D_EDGE, TE_STATS), lambda i: (0, i)),
            pl.BlockSpec((N // nsteps, D_IN), lambda i: (i, 0)),
            pl.BlockSpec((D_IN, DP), lambda i: (0, 0)),
            pl.BlockSpec((1, DP), lambda i: (0, 0)),
        ],
        out_specs=(
            pl.BlockSpec((D_EDGE + 1, D_EDGE + 1), lambda i: (0, 0)),
            pl.BlockSpec((N // nsteps, DP), lambda i: (i, 0)),
        ),
        out_shape=(
            jax.ShapeDtypeStruct((D_EDGE + 1, D_EDGE + 1), jnp.float32),
            jax.ShapeDtypeStruct((N, DP), jnp.float32),
        ),
    )(eat, x, root_p, bias_p)


def _gather_body(x_hbm, src_hbm, out_hbm, idx_v, rows_v, sem):
    c = lax.axis_index("c")
    s = lax.axis_index("s")
    wid = s * NC + c
    pltpu.sync_copy(src_hbm.at[pl.ds(wid * ROWS_W, ROWS_W)], idx_v)

    def body(j, carry):
        pltpu.async_copy(x_hbm.at[idx_v.at[j]], rows_v, sem).wait()
        pltpu.sync_copy(
            rows_v, out_hbm.at[pl.ds(wid * ROWS_W * CHUNK + j * CHUNK, CHUNK)])
        return carry

    lax.fori_loop(0, ROWS_W, body, 0)


def _gather_rows(x, src2d):
    kfn = pl.kernel(
        _gather_body,
        out_type=jax.ShapeDtypeStruct((E, D_IN), jnp.float32),
        mesh=plsc.VectorSubcoreMesh(core_axis_name="c", subcore_axis_name="s"),
        compiler_params=pltpu.CompilerParams(use_tc_tiling_on_sc=False),
        scratch_types=[
            pltpu.VMEM((ROWS_W, CHUNK), jnp.int32),
            pltpu.VMEM((CHUNK, D_IN), jnp.float32),
            pltpu.SemaphoreType.DMA,
        ],
    )
    return kfn(x, src2d)


def _msg_body(eat_ref, xs_ref, W1_ref, a_ref, c_ref, W2f_ref, S_ref, b2r_ref,
              m_ref):
    eat = eat_ref[...]
    xs = xs_ref[...]
    xsb = xs.astype(jnp.bfloat16)
    h = lax.dot_general(eat, W1_ref[...], (((0,), (0,)), ((), ())),
                        preferred_element_type=jnp.float32)
    h = jnp.maximum(h * a_ref[...] + c_ref[...], 0.0)
    h2 = jnp.concatenate([h, h], axis=1)
    m = jnp.dot(xsb, b2r_ref[...], preferred_element_type=jnp.float32)
    for j in range(D_OUT // 2):
        Gj = jnp.dot(xsb, W2f_ref[:, j * D_IN:(j + 1) * D_IN],
                     preferred_element_type=jnp.float32)
        prodj = (Gj * h2).astype(jnp.bfloat16)
        m = m + jnp.dot(prodj, S_ref[j * D_IN:(j + 1) * D_IN, :],
                        preferred_element_type=jnp.float32)
    m_ref[...] = m


def _messages(eat, x_src, W1, a_vec, c_vec, W2flat, S, b2r):
    full = lambda r, c: pl.BlockSpec((r, c), lambda i: (0, 0))
    return pl.pallas_call(
        _msg_body,
        grid=(E // TE_MSG,),
        in_specs=[
            pl.BlockSpec((D_EDGE, TE_MSG), lambda i: (0, i)),
            pl.BlockSpec((TE_MSG, D_IN), lambda i: (i, 0)),
            full(D_EDGE, H),
            full(1, H),
            full(1, H),
            full(D_IN, D_OUT * H),
            full(D_OUT * H, DP),
            full(D_IN, DP),
        ],
        out_specs=pl.BlockSpec((TE_MSG, DP), lambda i: (i, 0)),
        out_shape=jax.ShapeDtypeStruct((E, DP), jnp.float32),
    )(eat, x_src, W1, a_vec, c_vec, W2flat, S, b2r)


def _scatter_body(m_hbm, dst_hbm, zeros_hbm, out_hbm, idx_v, mbuf, aggr_sh,
                  sem):
    c = lax.axis_index("c")
    s = lax.axis_index("s")
    wid = s * NC + c
    pltpu.sync_copy(zeros_hbm.at[pl.ds(s * N_TILE, N_TILE)],
                    aggr_sh.at[pl.ds(s * N_TILE, N_TILE)])
    pltpu.sync_copy(dst_hbm.at[pl.ds(wid * ROWS_W, ROWS_W)], idx_v)
    plsc.subcore_barrier()

    def body(j, carry):
        pltpu.sync_copy(
            m_hbm.at[pl.ds(wid * ROWS_W * CHUNK + j * CHUNK, CHUNK)], mbuf)
        pltpu.sync_copy(mbuf, aggr_sh.at[idx_v.at[j]], add=True)
        return carry

    lax.fori_loop(0, ROWS_W, body, 0)
    plsc.subcore_barrier()
    pltpu.sync_copy(aggr_sh.at[pl.ds(s * N_TILE, N_TILE)],
                    out_hbm.at[c, pl.ds(s * N_TILE, N_TILE)])


def _scatter_add(m, dst2d, zeros):
    kfn = pl.kernel(
        _scatter_body,
        out_type=jax.ShapeDtypeStruct((NC, N_PAD, DP), jnp.float32),
        mesh=plsc.VectorSubcoreMesh(core_axis_name="c", subcore_axis_name="s"),
        compiler_params=pltpu.CompilerParams(use_tc_tiling_on_sc=False),
        scratch_types=[
            pltpu.VMEM((ROWS_W, CHUNK), jnp.int32),
            pltpu.VMEM((CHUNK, DP), jnp.float32),
            pltpu.VMEM_SHARED((N_PAD, DP), jnp.float32),
            pltpu.SemaphoreType.DMA,
        ],
    )
    return kfn(m, dst2d, zeros)


def _final_body(p0_ref, p1_ref, xr_ref, b_ref, A1_ref, bA1_ref, A2_ref,
                bA2_ref, out_ref, acc_ref):
    i = pl.program_id(0)
    out32 = p0_ref[...] + p1_ref[...] + xr_ref[...]
    lanes = lax.broadcasted_iota(jnp.int32, (TN_FIN, DP), 1)
    out_aug = out32 + (lanes == D_OUT).astype(jnp.float32)
    gids = lax.broadcasted_iota(jnp.int32, (TN_FIN, NG), 1)
    onehot = (b_ref[...] == gids).astype(jnp.float32)
    part = lax.dot_general(onehot, out_aug, (((0,), (0,)), ((), ())),
                           preferred_element_type=jnp.float32)

    @pl.when(i == 0)
    def _():
        acc_ref[...] = part

    @pl.when(i > 0)
    def _():
        acc_ref[...] = acc_ref[...] + part

    @pl.when(i == (N // TN_FIN) - 1)
    def _():
        P = acc_ref[...]
        cnt = P[:, D_OUT:D_OUT + 1]
        pooled = P / jnp.maximum(cnt, 1.0)
        z = jnp.maximum(
            jnp.dot(pooled, A1_ref[...], preferred_element_type=jnp.float32)
            + bA1_ref[...], 0.0)
        out_ref[...] = jnp.dot(
            z, A2_ref[...], preferred_element_type=jnp.float32) + bA2_ref[...]


def _finalize(p0, p1, xroot, batch2d, A1p, bA1, A2, bA2):
    full = lambda r, c: pl.BlockSpec((r, c), lambda i: (0, 0))
    return pl.pallas_call(
        _final_body,
        grid=(N // TN_FIN,),
        in_specs=[
            pl.BlockSpec((TN_FIN, DP), lambda i: (i, 0)),
            pl.BlockSpec((TN_FIN, DP), lambda i: (i, 0)),
            pl.BlockSpec((TN_FIN, DP), lambda i: (i, 0)),
            pl.BlockSpec((TN_FIN, 1), lambda i: (i, 0)),
            full(DP, HID),
            full(1, HID),
            full(HID, N_ACT),
            full(1, N_ACT),
        ],
        out_specs=pl.BlockSpec((NG, N_ACT), lambda i: (0, 0)),
        out_shape=jax.ShapeDtypeStruct((NG, N_ACT), jnp.float32),
        scratch_shapes=[pltpu.VMEM((NG, DP), jnp.float32)],
    )(p0, p1, xroot, batch2d, A1p, bA1, A2, bA2)


def kernel(x, edge_index, edge_attr, batch, W1, b1, gamma, beta, W2, b2, root,
           bias, A1, bA1, A2, bA2):
    f32 = jnp.float32
    src2d = edge_index[0].reshape(NW * ROWS_W, CHUNK)
    dst2d = edge_index[1].reshape(NW * ROWS_W, CHUNK)
    root_p = jnp.pad(root, ((0, 0), (0, DP - D_OUT)))
    bias_p = jnp.pad(bias, (0, DP - D_OUT)).reshape(1, DP)

    eat = edge_attr.T
    C_aug, xroot = _edge_stats(eat, x, root_p, bias_p)
    s_vec = C_aug[D_EDGE, :D_EDGE]
    Cm = C_aug[:D_EDGE, :D_EDGE]
    mu = (s_vec / E) @ W1 + b1
    Eh2 = (jnp.einsum("ij,ik,kj->j", W1, Cm, W1)
           + 2.0 * b1 * (s_vec @ W1)) / E + b1 * b1
    var = Eh2 - mu * mu
    inv = gamma * lax.rsqrt(var + EPS)
    a_vec = inv.reshape(1, H)
    c_vec = ((b1 - mu) * inv + beta).reshape(1, H)

    W2flat = W2.T.reshape(D_IN, D_OUT * H).astype(jnp.bfloat16)
    col = jnp.arange(D_OUT * H, dtype=jnp.int32)[:, None]
    S = (col // H == jnp.arange(DP, dtype=jnp.int32)[None, :]).astype(
        jnp.bfloat16)
    b2r = jnp.pad(b2.reshape(D_IN, D_OUT),
                  ((0, 0), (0, DP - D_OUT))).astype(jnp.bfloat16)
    A1p = jnp.pad(A1, ((0, DP - D_OUT), (0, 0)))

    x_src = _gather_rows(x, src2d)

    m4 = _messages(eat, x_src, W1, a_vec, c_vec, W2flat, S, b2r)

    partials = _scatter_add(m4, dst2d, jnp.zeros((N_PAD, DP), f32))

    return _finalize(partials[0, :N], partials[1, :N], xroot,
                     batch.reshape(N, 1), A1p, bA1.reshape(1, HID), A2,
                     bA2.reshape(1, N_ACT))


# speedup vs baseline: 1.1282x; 1.0044x over previous
import functools

import jax
import jax.numpy as jnp
from jax import lax
from jax.experimental import pallas as pl
from jax.experimental.pallas import tpu as pltpu
from jax.experimental.pallas import tpu_sc as plsc

N = 10000
E = 160000
D_IN = 128
D_OUT = 20
DP = 32
D_EDGE = 16
H = 64
NG = 64
N_ACT = 16
HID = 256
EPS = 1e-5

NC = 2
NS = 16
NW = NC * NS
CHUNK = 125
ROWS_W = E // NW // CHUNK
CHUNK_S = 40
ROWS_W_S = E // NW // CHUNK_S
N_PAD = 10240
N_TILE = N_PAD // NS

TE_STATS = 6400
TE_MSG = 3200
E_HALF = E // 2
ROWS_H = ROWS_W // 2
TN_FIN = 2000


def _stats_body(eat_ref, x_ref, root_ref, bias_ref, out_ref, xr_ref):
    i = pl.program_id(0)
    eat = eat_ref[...]
    aug = jnp.concatenate(
        [eat, jnp.ones((1, eat.shape[1]), jnp.float32)], axis=0)
    part = lax.dot_general(aug, aug, (((1,), (1,)), ((), ())),
                           preferred_element_type=jnp.float32)

    @pl.when(i == 0)
    def _():
        out_ref[...] = part

    @pl.when(i > 0)
    def _():
        out_ref[...] = out_ref[...] + part

    xr_ref[...] = jnp.dot(x_ref[...], root_ref[...],
                          preferred_element_type=jnp.float32) + bias_ref[...]


def _edge_stats(eat, x, root_p, bias_p):
    nsteps = E // TE_STATS
    return pl.pallas_call(
        _stats_body,
        grid=(nsteps,),
        in_specs=[
            pl.BlockSpec((D_EDGE, TE_STATS), lambda i: (0, i)),
            pl.BlockSpec((N // nsteps, D_IN), lambda i: (i, 0)),
            pl.BlockSpec((D_IN, DP), lambda i: (0, 0)),
            pl.BlockSpec((1, DP), lambda i: (0, 0)),
        ],
        out_specs=(
            pl.BlockSpec((D_EDGE + 1, D_EDGE + 1), lambda i: (0, 0)),
            pl.BlockSpec((N // nsteps, DP), lambda i: (i, 0)),
        ),
        out_shape=(
            jax.ShapeDtypeStruct((D_EDGE + 1, D_EDGE + 1), jnp.float32),
            jax.ShapeDtypeStruct((N, DP), jnp.float32),
        ),
    )(eat, x, root_p, bias_p)


def _gather_body(x_hbm, src_hbm, out_hbm, idx_v, rows_v, sem):
    c = lax.axis_index("c")
    s = lax.axis_index("s")
    wid = s * NC + c
    pltpu.sync_copy(src_hbm.at[pl.ds(wid * ROWS_W, ROWS_W)], idx_v)

    def body(j, carry):
        pltpu.async_copy(x_hbm.at[idx_v.at[j]], rows_v, sem).wait()
        pltpu.sync_copy(
            rows_v, out_hbm.at[pl.ds(wid * ROWS_W * CHUNK + j * CHUNK, CHUNK)])
        return carry

    lax.fori_loop(0, ROWS_W, body, 0)


def _gather_rows(x, src2d):
    kfn = pl.kernel(
        _gather_body,
        out_type=jax.ShapeDtypeStruct((E, D_IN), jnp.float32),
        mesh=plsc.VectorSubcoreMesh(core_axis_name="c", subcore_axis_name="s"),
        compiler_params=pltpu.CompilerParams(use_tc_tiling_on_sc=False),
        scratch_types=[
            pltpu.VMEM((ROWS_W, CHUNK), jnp.int32),
            pltpu.VMEM((CHUNK, D_IN), jnp.float32),
            pltpu.SemaphoreType.DMA,
        ],
    )
    return kfn(x, src2d)


def _msg_body(eat_ref, xs_ref, W1_ref, a_ref, c_ref, W2f_ref, S_ref, b2r_ref,
              m_ref):
    eat = eat_ref[...]
    xs = xs_ref[...]
    xsb = xs.astype(jnp.bfloat16)
    h = lax.dot_general(eat, W1_ref[...], (((0,), (0,)), ((), ())),
                        preferred_element_type=jnp.float32)
    h = jnp.maximum(h * a_ref[...] + c_ref[...], 0.0)
    h2 = jnp.concatenate([h, h], axis=1)
    m = jnp.dot(xsb, b2r_ref[...], preferred_element_type=jnp.float32)
    for j in range(D_OUT // 2):
        Gj = jnp.dot(xsb, W2f_ref[:, j * D_IN:(j + 1) * D_IN],
                     preferred_element_type=jnp.float32)
        prodj = (Gj * h2).astype(jnp.bfloat16)
        m = m + jnp.dot(prodj, S_ref[j * D_IN:(j + 1) * D_IN, :],
                        preferred_element_type=jnp.float32)
    m_ref[...] = m


def _messages(eat, x_src, W1, a_vec, c_vec, W2flat, S, b2r, half):
    full = lambda r, c: pl.BlockSpec((r, c), lambda i: (0, 0))
    off = half * (E_HALF // TE_MSG)
    return pl.pallas_call(
        _msg_body,
        grid=(E_HALF // TE_MSG,),
        in_specs=[
            pl.BlockSpec((D_EDGE, TE_MSG), lambda i: (0, i + off)),
            pl.BlockSpec((TE_MSG, D_IN), lambda i: (i + off, 0)),
            full(D_EDGE, H),
            full(1, H),
            full(1, H),
            full(D_IN, D_OUT * H),
            full(D_OUT * H, DP),
            full(D_IN, DP),
        ],
        out_specs=pl.BlockSpec((TE_MSG, DP), lambda i: (i, 0)),
        out_shape=jax.ShapeDtypeStruct((E_HALF, DP), jnp.float32),
    )(eat, x_src, W1, a_vec, c_vec, W2flat, S, b2r)


def _scatter_body(m_hbm, dst_hbm, init_hbm, out_hbm, idx_v, mbuf, aggr_sh,
                  sem):
    c = lax.axis_index("c")
    s = lax.axis_index("s")
    wid = s * NC + c
    pltpu.sync_copy(init_hbm.at[c, pl.ds(s * N_TILE, N_TILE)],
                    aggr_sh.at[pl.ds(s * N_TILE, N_TILE)])
    pltpu.sync_copy(dst_hbm.at[pl.ds(wid * ROWS_H, ROWS_H)], idx_v)
    plsc.subcore_barrier()

    def body(j, carry):
        pltpu.sync_copy(
            m_hbm.at[pl.ds(wid * ROWS_H * CHUNK + j * CHUNK, CHUNK)], mbuf)
        pltpu.sync_copy(mbuf, aggr_sh.at[idx_v.at[j]], add=True)
        return carry

    lax.fori_loop(0, ROWS_H, body, 0)
    plsc.subcore_barrier()
    pltpu.sync_copy(aggr_sh.at[pl.ds(s * N_TILE, N_TILE)],
                    out_hbm.at[c, pl.ds(s * N_TILE, N_TILE)])


def _scatter_add(m, dst2d, init):
    kfn = pl.kernel(
        _scatter_body,
        out_type=jax.ShapeDtypeStruct((NC, N_PAD, DP), jnp.float32),
        mesh=plsc.VectorSubcoreMesh(core_axis_name="c", subcore_axis_name="s"),
        compiler_params=pltpu.CompilerParams(use_tc_tiling_on_sc=False),
        scratch_types=[
            pltpu.VMEM((ROWS_H, CHUNK), jnp.int32),
            pltpu.VMEM((CHUNK, DP), jnp.float32),
            pltpu.VMEM_SHARED((N_PAD, DP), jnp.float32),
            pltpu.SemaphoreType.DMA,
        ],
    )
    return kfn(m, dst2d, init)


def _final_body(p0_ref, p1_ref, xr_ref, b_ref, A1_ref, bA1_ref, A2_ref,
                bA2_ref, out_ref, acc_ref):
    i = pl.program_id(0)
    out32 = p0_ref[...] + p1_ref[...] + xr_ref[...]
    lanes = lax.broadcasted_iota(jnp.int32, (TN_FIN, DP), 1)
    out_aug = out32 + (lanes == D_OUT).astype(jnp.float32)
    gids = lax.broadcasted_iota(jnp.int32, (TN_FIN, NG), 1)
    onehot = (b_ref[...] == gids).astype(jnp.float32)
    part = lax.dot_general(onehot, out_aug, (((0,), (0,)), ((), ())),
                           preferred_element_type=jnp.float32)

    @pl.when(i == 0)
    def _():
        acc_ref[...] = part

    @pl.when(i > 0)
    def _():
        acc_ref[...] = acc_ref[...] + part

    @pl.when(i == (N // TN_FIN) - 1)
    def _():
        P = acc_ref[...]
        cnt = P[:, D_OUT:D_OUT + 1]
        pooled = P / jnp.maximum(cnt, 1.0)
        z = jnp.maximum(
            jnp.dot(pooled, A1_ref[...], preferred_element_type=jnp.float32)
            + bA1_ref[...], 0.0)
        out_ref[...] = jnp.dot(
            z, A2_ref[...], preferred_element_type=jnp.float32) + bA2_ref[...]


def _finalize(p0, p1, xroot, batch2d, A1p, bA1, A2, bA2):
    full = lambda r, c: pl.BlockSpec((r, c), lambda i: (0, 0))
    return pl.pallas_call(
        _final_body,
        grid=(N // TN_FIN,),
        in_specs=[
            pl.BlockSpec((TN_FIN, DP), lambda i: (i, 0)),
            pl.BlockSpec((TN_FIN, DP), lambda i: (i, 0)),
            pl.BlockSpec((TN_FIN, DP), lambda i: (i, 0)),
            pl.BlockSpec((TN_FIN, 1), lambda i: (i, 0)),
            full(DP, HID),
            full(1, HID),
            full(HID, N_ACT),
            full(1, N_ACT),
        ],
        out_specs=pl.BlockSpec((NG, N_ACT), lambda i: (0, 0)),
        out_shape=jax.ShapeDtypeStruct((NG, N_ACT), jnp.float32),
        scratch_shapes=[pltpu.VMEM((NG, DP), jnp.float32)],
    )(p0, p1, xroot, batch2d, A1p, bA1, A2, bA2)


def kernel(x, edge_index, edge_attr, batch, W1, b1, gamma, beta, W2, b2, root,
           bias, A1, bA1, A2, bA2):
    f32 = jnp.float32
    src2d = edge_index[0].reshape(NW * ROWS_W, CHUNK)
    dst2d = edge_index[1].reshape(NW * ROWS_W, CHUNK)
    root_p = jnp.pad(root, ((0, 0), (0, DP - D_OUT)))
    bias_p = jnp.pad(bias, (0, DP - D_OUT)).reshape(1, DP)

    eat = edge_attr.T
    C_aug, xroot = _edge_stats(eat, x, root_p, bias_p)
    s_vec = C_aug[D_EDGE, :D_EDGE]
    Cm = C_aug[:D_EDGE, :D_EDGE]
    mu = (s_vec / E) @ W1 + b1
    Eh2 = (jnp.einsum("ij,ik,kj->j", W1, Cm, W1)
           + 2.0 * b1 * (s_vec @ W1)) / E + b1 * b1
    var = Eh2 - mu * mu
    inv = gamma * lax.rsqrt(var + EPS)
    a_vec = inv.reshape(1, H)
    c_vec = ((b1 - mu) * inv + beta).reshape(1, H)

    W2flat = W2.T.reshape(D_IN, D_OUT * H).astype(jnp.bfloat16)
    col = jnp.arange(D_OUT * H, dtype=jnp.int32)[:, None]
    S = (col // H == jnp.arange(DP, dtype=jnp.int32)[None, :]).astype(
        jnp.bfloat16)
    b2r = jnp.pad(b2.reshape(D_IN, D_OUT),
                  ((0, 0), (0, DP - D_OUT))).astype(jnp.bfloat16)
    A1p = jnp.pad(A1, ((0, DP - D_OUT), (0, 0)))

    x_src = _gather_rows(x, src2d)

    m_a = _messages(eat, x_src, W1, a_vec, c_vec, W2flat, S, b2r, 0)
    m_b = _messages(eat, x_src, W1, a_vec, c_vec, W2flat, S, b2r, 1)

    nrow_h = NW * ROWS_H
    p_a = _scatter_add(m_a, dst2d[:nrow_h], jnp.zeros((NC, N_PAD, DP), f32))
    partials = _scatter_add(m_b, dst2d[nrow_h:], p_a)

    return _finalize(partials[0, :N], partials[1, :N], xroot,
                     batch.reshape(N, 1), A1p, bA1.reshape(1, HID), A2,
                     bA2.reshape(1, N_ACT))


# speedup vs baseline: 1.1283x; 1.0001x over previous
import jax
import jax.numpy as jnp
from jax import lax
from jax.experimental import pallas as pl
from jax.experimental.pallas import tpu as pltpu
from jax.experimental.pallas import tpu_sc as plsc

N = 10000
E = 160000
D_IN = 128
D_OUT = 20
DP = 32
D_EDGE = 16
H = 64
NG = 64
N_ACT = 16
HID = 256
EPS = 1e-5

NC = 2
NS = 16
NW = NC * NS
CHUNK = 125
ROWS_W = E // NW // CHUNK
N_PAD = 10240
N_TILE = N_PAD // NS

TE_STATS = 6400
TE_MSG = 3200
E_HALF = E // 2
ROWS_H = ROWS_W // 2
TN_FIN = 2000


def _stats_body(eat_ref, x_ref, root_ref, bias_ref, out_ref, xr_ref):
    i = pl.program_id(0)
    eat = eat_ref[...]
    aug = jnp.concatenate(
        [eat, jnp.ones((1, eat.shape[1]), jnp.float32)], axis=0)
    part = lax.dot_general(aug, aug, (((1,), (1,)), ((), ())),
                           preferred_element_type=jnp.float32)

    @pl.when(i == 0)
    def _():
        out_ref[...] = part

    @pl.when(i > 0)
    def _():
        out_ref[...] = out_ref[...] + part

    xr_ref[...] = jnp.dot(x_ref[...], root_ref[...],
                          preferred_element_type=jnp.float32) + bias_ref[...]


def _edge_stats(eat, x, root_p, bias_p):
    nsteps = E // TE_STATS
    return pl.pallas_call(
        _stats_body,
        grid=(nsteps,),
        in_specs=[
            pl.BlockSpec((D_EDGE, TE_STATS), lambda i: (0, i)),
            pl.BlockSpec((N // nsteps, D_IN), lambda i: (i, 0)),
            pl.BlockSpec((D_IN, DP), lambda i: (0, 0)),
            pl.BlockSpec((1, DP), lambda i: (0, 0)),
        ],
        out_specs=(
            pl.BlockSpec((D_EDGE + 1, D_EDGE + 1), lambda i: (0, 0)),
            pl.BlockSpec((N // nsteps, DP), lambda i: (i, 0)),
        ),
        out_shape=(
            jax.ShapeDtypeStruct((D_EDGE + 1, D_EDGE + 1), jnp.float32),
            jax.ShapeDtypeStruct((N, DP), jnp.float32),
        ),
    )(eat, x, root_p, bias_p)


def _gather_body(x_hbm, src_hbm, out_hbm, idx_v, rows_v, sem):
    c = lax.axis_index("c")
    s = lax.axis_index("s")
    wid = s * NC + c
    pltpu.sync_copy(src_hbm.at[pl.ds(wid * ROWS_W, ROWS_W)], idx_v)

    def body(j, carry):
        pltpu.async_copy(x_hbm.at[idx_v.at[j]], rows_v, sem).wait()
        pltpu.sync_copy(
            rows_v, out_hbm.at[pl.ds(wid * ROWS_W * CHUNK + j * CHUNK, CHUNK)])
        return carry

    lax.fori_loop(0, ROWS_W, body, 0)


def _gather_rows(x, src2d):
    kfn = pl.kernel(
        _gather_body,
        out_type=jax.ShapeDtypeStruct((E, D_IN), jnp.float32),
        mesh=plsc.VectorSubcoreMesh(core_axis_name="c", subcore_axis_name="s"),
        compiler_params=pltpu.CompilerParams(use_tc_tiling_on_sc=False),
        scratch_types=[
            pltpu.VMEM((ROWS_W, CHUNK), jnp.int32),
            pltpu.VMEM((CHUNK, D_IN), jnp.float32),
            pltpu.SemaphoreType.DMA,
        ],
    )
    return kfn(x, src2d)


def _msg_body(eat_ref, xs_ref, W1_ref, a_ref, c_ref, W2f_ref, S_ref, b2r_ref,
              m_ref):
    eat = eat_ref[...]
    xs = xs_ref[...]
    xsb = xs.astype(jnp.bfloat16)
    h = lax.dot_general(eat, W1_ref[...], (((0,), (0,)), ((), ())),
                        preferred_element_type=jnp.float32)
    h = jnp.maximum(h * a_ref[...] + c_ref[...], 0.0)
    h2 = jnp.concatenate([h, h], axis=1)
    m = jnp.dot(xsb, b2r_ref[...], preferred_element_type=jnp.float32)
    for j in range(D_OUT // 2):
        Gj = jnp.dot(xsb, W2f_ref[:, j * D_IN:(j + 1) * D_IN],
                     preferred_element_type=jnp.float32)
        prodj = (Gj * h2).astype(jnp.bfloat16)
        m = m + jnp.dot(prodj, S_ref[j * D_IN:(j + 1) * D_IN, :],
                        preferred_element_type=jnp.float32)
    m_ref[...] = m


def _messages(eat, x_src, W1, a_vec, c_vec, W2flat, S, b2r, half):
    full = lambda r, c: pl.BlockSpec((r, c), lambda i: (0, 0))
    off = half * (E_HALF // TE_MSG)
    return pl.pallas_call(
        _msg_body,
        grid=(E_HALF // TE_MSG,),
        in_specs=[
            pl.BlockSpec((D_EDGE, TE_MSG), lambda i: (0, i + off)),
            pl.BlockSpec((TE_MSG, D_IN), lambda i: (i + off, 0)),
            full(D_EDGE, H),
            full(1, H),
            full(1, H),
            full(D_IN, D_OUT * H),
            full(D_OUT * H, DP),
            full(D_IN, DP),
        ],
        out_specs=pl.BlockSpec((TE_MSG, DP), lambda i: (i, 0)),
        out_shape=jax.ShapeDtypeStruct((E_HALF, DP), jnp.float32),
    )(eat, x_src, W1, a_vec, c_vec, W2flat, S, b2r)


def _scatter_body(m_hbm, dst_hbm, init_hbm, out_hbm, idx_v, mbuf, aggr_sh,
                  sem):
    c = lax.axis_index("c")
    s = lax.axis_index("s")
    wid = s * NC + c
    pltpu.sync_copy(init_hbm.at[c, pl.ds(s * N_TILE, N_TILE)],
                    aggr_sh.at[pl.ds(s * N_TILE, N_TILE)])
    pltpu.sync_copy(dst_hbm.at[pl.ds(wid * ROWS_H, ROWS_H)], idx_v)
    plsc.subcore_barrier()

    def body(j, carry):
        pltpu.sync_copy(
            m_hbm.at[pl.ds(wid * ROWS_H * CHUNK + j * CHUNK, CHUNK)], mbuf)
        pltpu.sync_copy(mbuf, aggr_sh.at[idx_v.at[j]], add=True)
        return carry

    lax.fori_loop(0, ROWS_H, body, 0)
    plsc.subcore_barrier()
    pltpu.sync_copy(aggr_sh.at[pl.ds(s * N_TILE, N_TILE)],
                    out_hbm.at[c, pl.ds(s * N_TILE, N_TILE)])


def _scatter_add(m, dst2d, init):
    kfn = pl.kernel(
        _scatter_body,
        out_type=jax.ShapeDtypeStruct((NC, N_PAD, DP), jnp.float32),
        mesh=plsc.VectorSubcoreMesh(core_axis_name="c", subcore_axis_name="s"),
        compiler_params=pltpu.CompilerParams(use_tc_tiling_on_sc=False),
        scratch_types=[
            pltpu.VMEM((ROWS_H, CHUNK), jnp.int32),
            pltpu.VMEM((CHUNK, DP), jnp.float32),
            pltpu.VMEM_SHARED((N_PAD, DP), jnp.float32),
            pltpu.SemaphoreType.DMA,
        ],
    )
    return kfn(m, dst2d, init)


def _final_body(p0_ref, p1_ref, xr_ref, b_ref, A1_ref, bA1_ref, A2_ref,
                bA2_ref, out_ref, acc_ref):
    i = pl.program_id(0)
    out32 = p0_ref[...] + p1_ref[...] + xr_ref[...]
    lanes = lax.broadcasted_iota(jnp.int32, (TN_FIN, DP), 1)
    out_aug = out32 + (lanes == D_OUT).astype(jnp.float32)
    gids = lax.broadcasted_iota(jnp.int32, (TN_FIN, NG), 1)
    onehot = (b_ref[...] == gids).astype(jnp.float32)
    part = lax.dot_general(onehot, out_aug, (((0,), (0,)), ((), ())),
                           preferred_element_type=jnp.float32)

    @pl.when(i == 0)
    def _():
        acc_ref[...] = part

    @pl.when(i > 0)
    def _():
        acc_ref[...] = acc_ref[...] + part

    @pl.when(i == (N // TN_FIN) - 1)
    def _():
        P = acc_ref[...]
        cnt = P[:, D_OUT:D_OUT + 1]
        pooled = P / jnp.maximum(cnt, 1.0)
        z = jnp.maximum(
            jnp.dot(pooled, A1_ref[...], preferred_element_type=jnp.float32)
            + bA1_ref[...], 0.0)
        out_ref[...] = jnp.dot(
            z, A2_ref[...], preferred_element_type=jnp.float32) + bA2_ref[...]


def _finalize(p0, p1, xroot, batch2d, A1p, bA1, A2, bA2):
    full = lambda r, c: pl.BlockSpec((r, c), lambda i: (0, 0))
    return pl.pallas_call(
        _final_body,
        grid=(N // TN_FIN,),
        in_specs=[
            pl.BlockSpec((TN_FIN, DP), lambda i: (i, 0)),
            pl.BlockSpec((TN_FIN, DP), lambda i: (i, 0)),
            pl.BlockSpec((TN_FIN, DP), lambda i: (i, 0)),
            pl.BlockSpec((TN_FIN, 1), lambda i: (i, 0)),
            full(DP, HID),
            full(1, HID),
            full(HID, N_ACT),
            full(1, N_ACT),
        ],
        out_specs=pl.BlockSpec((NG, N_ACT), lambda i: (0, 0)),
        out_shape=jax.ShapeDtypeStruct((NG, N_ACT), jnp.float32),
        scratch_shapes=[pltpu.VMEM((NG, DP), jnp.float32)],
    )(p0, p1, xroot, batch2d, A1p, bA1, A2, bA2)


def kernel(x, edge_index, edge_attr, batch, W1, b1, gamma, beta, W2, b2, root,
           bias, A1, bA1, A2, bA2):
    f32 = jnp.float32
    src2d = edge_index[0].reshape(NW * ROWS_W, CHUNK)
    dst2d = edge_index[1].reshape(NW * ROWS_W, CHUNK)
    root_p = jnp.pad(root, ((0, 0), (0, DP - D_OUT)))
    bias_p = jnp.pad(bias, (0, DP - D_OUT)).reshape(1, DP)

    eat = edge_attr.T
    C_aug, xroot = _edge_stats(eat, x, root_p, bias_p)
    s_vec = C_aug[D_EDGE, :D_EDGE]
    Cm = C_aug[:D_EDGE, :D_EDGE]
    mu = (s_vec / E) @ W1 + b1
    Eh2 = (jnp.einsum("ij,ik,kj->j", W1, Cm, W1)
           + 2.0 * b1 * (s_vec @ W1)) / E + b1 * b1
    var = Eh2 - mu * mu
    inv = gamma * lax.rsqrt(var + EPS)
    a_vec = inv.reshape(1, H)
    c_vec = ((b1 - mu) * inv + beta).reshape(1, H)

    W2flat = W2.T.reshape(D_IN, D_OUT * H).astype(jnp.bfloat16)
    col = jnp.arange(D_OUT * H, dtype=jnp.int32)[:, None]
    S = (col // H == jnp.arange(DP, dtype=jnp.int32)[None, :]).astype(
        jnp.bfloat16)
    b2r = jnp.pad(b2.reshape(D_IN, D_OUT),
                  ((0, 0), (0, DP - D_OUT))).astype(jnp.bfloat16)
    A1p = jnp.pad(A1, ((0, DP - D_OUT), (0, 0)))

    x_src = _gather_rows(x, src2d)

    m_a = _messages(eat, x_src, W1, a_vec, c_vec, W2flat, S, b2r, 0)
    m_b = _messages(eat, x_src, W1, a_vec, c_vec, W2flat, S, b2r, 1)

    nrow_h = NW * ROWS_H
    p_a = _scatter_add(m_a, dst2d[:nrow_h], jnp.zeros((NC, N_PAD, DP), f32))
    partials = _scatter_add(m_b, dst2d[nrow_h:], p_a)

    return _finalize(partials[0, :N], partials[1, :N], xroot,
                     batch.reshape(N, 1), A1p, bA1.reshape(1, HID), A2,
                     bA2.reshape(1, N_ACT))
